# baseline (speedup 1.0000x reference)
import jax
import jax.numpy as jnp
from jax import lax
from jax.experimental import pallas as pl
from jax.experimental.pallas import tpu as pltpu

N = 10000
E = 160000
CHUNK = 2000
NCHUNK = E // CHUNK
HIGH = lax.Precision.HIGHEST


def _mp_body(src_ref, dst_ref, ea_ref, h_ref, h2_ref, mol_ref,
             m_ref, bet_ref, c_ref,
             rw1_ref, rw2_ref, rb_ref,
             f1w_ref, f1b_ref, f2w_ref, f2b_ref, f3w_ref, f3b_ref,
             out_ref, h_state, b_scratch):
    step = pl.program_id(0)

    @pl.when(step == 0)
    def _init():
        h_state[...] = h_ref[...]

    M = m_ref[0]
    BeT = bet_ref[0]
    c = c_ref[0]

    b_scratch[...] = lax.dot(ea_ref[0], BeT, precision=HIGH) + c

    def body(i, carry):
        s = src_ref[0, 0, i]
        d = dst_ref[0, 0, i]
        hs = h_state[pl.ds(s, 1), :]
        hd = h_state[pl.ds(d, 1), :]
        bi = b_scratch[pl.ds(i, 1), :]
        hcat = jnp.concatenate([hd, hs], axis=1)
        new = lax.dot(hcat, M) + bi
        h_state[pl.ds(d, 1), :] = new
        return carry

    lax.fori_loop(0, CHUNK, body, 0)

    @pl.when(step == 2 * NCHUNK - 1)
    def _tail():
        inv_n = jnp.float32(1.0 / N)
        hbar = jnp.sum(h_state[...], axis=0, keepdims=True) * inv_n
        h2bar = jnp.sum(h2_ref[...], axis=0, keepdims=True) * inv_n
        r32 = (lax.dot(hbar, rw1_ref[...], precision=HIGH)
               + lax.dot(h2bar, rw2_ref[...], precision=HIGH) + rb_ref[...])
        x = jnp.concatenate([r32, mol_ref[...]], axis=1)
        x = jnp.maximum(x, 0.0)
        x = jnp.maximum(lax.dot(x, f1w_ref[...], precision=HIGH) + f1b_ref[...], 0.0)
        x = jnp.maximum(lax.dot(x, f2w_ref[...], precision=HIGH) + f2b_ref[...], 0.0)
        out_ref[...] = lax.dot(x, f3w_ref[...], precision=HIGH) + f3b_ref[...]


def kernel(molfeats, edge_index, edge_attr, h, h2,
           V1_w, V1_b, E1_w, E1_b, U1_w, U1_b,
           V2_w, V2_b, E2_w, E2_b, U2_w, U2_b,
           R_w, R_b, fc1_w, fc1_b, fc2_w, fc2_b, fc3_w, fc3_b):
    src = edge_index[0]
    dst = edge_index[1]

    def prep(Vw, Vb, Ew, Eb, Uw, Ub):
        A = Uw[:, 0:5]
        Wm = Uw[:, 5:10]
        We = Uw[:, 10:21]
        M = jnp.concatenate([A.T, (Wm @ Vw).T], axis=0)
        BeT = (We @ Ew).T
        c = Wm @ Vb + We @ Eb + Ub
        return M, BeT, c.reshape(1, 5)

    M1, BeT1, c1 = prep(V1_w, V1_b, E1_w, E1_b, U1_w, U1_b)
    M2, BeT2, c2 = prep(V2_w, V2_b, E2_w, E2_b, U2_w, U2_b)
    Ms = jnp.stack([M1, M2])
    BeTs = jnp.stack([BeT1, BeT2])
    cs = jnp.stack([c1, c2])

    src3 = src.reshape(NCHUNK, 1, CHUNK)
    dst3 = dst.reshape(NCHUNK, 1, CHUNK)
    ea3 = edge_attr.reshape(NCHUNK, CHUNK, 11)
    mol2 = molfeats.reshape(1, 202)
    RwT = R_w.T
    Rw1T = RwT[:5]
    Rw2T = RwT[5:]
    Rb2 = R_b.reshape(1, 32)
    f1w = fc1_w.T
    f1b = fc1_b.reshape(1, 128)
    f2w = fc2_w.T
    f2b = fc2_b.reshape(1, 32)
    f3w = fc3_w.T
    f3b = fc3_b.reshape(1, 1)

    grid = (2 * NCHUNK,)

    def chunk_map(i):
        return (i % NCHUNK, 0, 0)

    def layer_map(i):
        return (i // NCHUNK, 0, 0)

    const2 = lambda i: (0, 0)

    out = pl.pallas_call(
        _mp_body,
        grid=grid,
        in_specs=[
            pl.BlockSpec((1, 1, CHUNK), chunk_map, memory_space=pltpu.SMEM),
            pl.BlockSpec((1, 1, CHUNK), chunk_map, memory_space=pltpu.SMEM),
            pl.BlockSpec((1, CHUNK, 11), chunk_map),
            pl.BlockSpec((N, 5), const2),
            pl.BlockSpec((N, 5), const2),
            pl.BlockSpec((1, 202), const2),
            pl.BlockSpec((1, 10, 5), layer_map),
            pl.BlockSpec((1, 11, 5), layer_map),
            pl.BlockSpec((1, 1, 5), layer_map),
            pl.BlockSpec((5, 32), const2),
            pl.BlockSpec((5, 32), const2),
            pl.BlockSpec((1, 32), const2),
            pl.BlockSpec((234, 128), const2),
            pl.BlockSpec((1, 128), const2),
            pl.BlockSpec((128, 32), const2),
            pl.BlockSpec((1, 32), const2),
            pl.BlockSpec((32, 1), const2),
            pl.BlockSpec((1, 1), const2),
        ],
        out_specs=pl.BlockSpec((1, 1), const2),
        out_shape=jax.ShapeDtypeStruct((1, 1), jnp.float32),
        scratch_shapes=[
            pltpu.VMEM((N, 5), jnp.float32),
            pltpu.VMEM((CHUNK, 5), jnp.float32),
        ],
    )(src3, dst3, ea3, h, h2, mol2, Ms, BeTs, cs,
      Rw1T, Rw2T, Rb2, f1w, f1b, f2w, f2b, f3w, f3b)
    return out.reshape(1)


# speedup vs baseline: 2.1701x; 2.1701x over previous
import jax
import jax.numpy as jnp
from jax import lax
from jax.experimental import pallas as pl
from jax.experimental.pallas import tpu as pltpu

N = 10000
E = 160000
CHUNK = 2000
NCHUNK = E // CHUNK
HIGH = lax.Precision.HIGHEST


def _mp_body(src_ref, dst_ref, ea_ref, h_ref, h2_ref, mol_ref,
             m_ref, bet_ref, c_ref,
             rw1_ref, rw2_ref, rb_ref,
             f1w_ref, f1b_ref, f2w_ref, f2b_ref, f3w_ref, f3b_ref,
             out_ref, h_state, b_scratch):
    step = pl.program_id(0)

    @pl.when(step == 0)
    def _init():
        h_state[...] = h_ref[...]

    M = m_ref[0]
    BeT = bet_ref[0]
    c = c_ref[0]

    b_scratch[...] = lax.dot(ea_ref[0], BeT, precision=HIGH) + c

    def group_body(carry):
        i0 = carry[0]
        d = dst_ref[0, 0, i0]
        hc0 = h_state[pl.ds(d, 1), :]

        def inner_cond(c):
            i2, _ = c
            i2c = jnp.minimum(i2, CHUNK - 1)
            return (i2 < CHUNK) & (dst_ref[0, 0, i2c] == d)

        def inner_body(c):
            i2, hc = c
            s = src_ref[0, 0, i2]
            hs_mem = h_state[pl.ds(s, 1), :]
            hs = jnp.where(s == d, hc, hs_mem)
            acc = b_scratch[pl.ds(i2, 1), :]
            for m in range(5):
                acc = acc + hc[:, m:m + 1] * M[m:m + 1, :]
                acc = acc + hs[:, m:m + 1] * M[m + 5:m + 6, :]
            return (i2 + 1, acc)

        i_end, hc_fin = lax.while_loop(inner_cond, inner_body, (i0, hc0))
        h_state[pl.ds(d, 1), :] = hc_fin
        return (i_end,)

    lax.while_loop(lambda c: c[0] < CHUNK, group_body, (jnp.int32(0),))

    @pl.when(step == 2 * NCHUNK - 1)
    def _tail():
        inv_n = jnp.float32(1.0 / N)
        hbar = jnp.sum(h_state[...], axis=0, keepdims=True) * inv_n
        h2bar = jnp.sum(h2_ref[...], axis=0, keepdims=True) * inv_n
        r32 = (lax.dot(hbar, rw1_ref[...], precision=HIGH)
               + lax.dot(h2bar, rw2_ref[...], precision=HIGH) + rb_ref[...])
        x = jnp.concatenate([r32, mol_ref[...]], axis=1)
        x = jnp.maximum(x, 0.0)
        x = jnp.maximum(lax.dot(x, f1w_ref[...], precision=HIGH) + f1b_ref[...], 0.0)
        x = jnp.maximum(lax.dot(x, f2w_ref[...], precision=HIGH) + f2b_ref[...], 0.0)
        out_ref[...] = lax.dot(x, f3w_ref[...], precision=HIGH) + f3b_ref[...]


def kernel(molfeats, edge_index, edge_attr, h, h2,
           V1_w, V1_b, E1_w, E1_b, U1_w, U1_b,
           V2_w, V2_b, E2_w, E2_b, U2_w, U2_b,
           R_w, R_b, fc1_w, fc1_b, fc2_w, fc2_b, fc3_w, fc3_b):
    src = edge_index[0]
    dst = edge_index[1]

    def prep(Vw, Vb, Ew, Eb, Uw, Ub):
        A = Uw[:, 0:5]
        Wm = Uw[:, 5:10]
        We = Uw[:, 10:21]
        M = jnp.concatenate([A.T, (Wm @ Vw).T], axis=0)
        BeT = (We @ Ew).T
        c = Wm @ Vb + We @ Eb + Ub
        return M, BeT, c.reshape(1, 5)

    M1, BeT1, c1 = prep(V1_w, V1_b, E1_w, E1_b, U1_w, U1_b)
    M2, BeT2, c2 = prep(V2_w, V2_b, E2_w, E2_b, U2_w, U2_b)
    Ms = jnp.stack([M1, M2])
    BeTs = jnp.stack([BeT1, BeT2])
    cs = jnp.stack([c1, c2])

    src3 = src.reshape(NCHUNK, 1, CHUNK)
    dst3 = dst.reshape(NCHUNK, 1, CHUNK)
    ea3 = edge_attr.reshape(NCHUNK, CHUNK, 11)
    mol2 = molfeats.reshape(1, 202)
    RwT = R_w.T
    Rw1T = RwT[:5]
    Rw2T = RwT[5:]
    Rb2 = R_b.reshape(1, 32)
    f1w = fc1_w.T
    f1b = fc1_b.reshape(1, 128)
    f2w = fc2_w.T
    f2b = fc2_b.reshape(1, 32)
    f3w = fc3_w.T
    f3b = fc3_b.reshape(1, 1)

    grid = (2 * NCHUNK,)

    def chunk_map(i):
        return (i % NCHUNK, 0, 0)

    def layer_map(i):
        return (i // NCHUNK, 0, 0)

    const2 = lambda i: (0, 0)

    out = pl.pallas_call(
        _mp_body,
        grid=grid,
        in_specs=[
            pl.BlockSpec((1, 1, CHUNK), chunk_map, memory_space=pltpu.SMEM),
            pl.BlockSpec((1, 1, CHUNK), chunk_map, memory_space=pltpu.SMEM),
            pl.BlockSpec((1, CHUNK, 11), chunk_map),
            pl.BlockSpec((N, 5), const2),
            pl.BlockSpec((N, 5), const2),
            pl.BlockSpec((1, 202), const2),
            pl.BlockSpec((1, 10, 5), layer_map),
            pl.BlockSpec((1, 11, 5), layer_map),
            pl.BlockSpec((1, 1, 5), layer_map),
            pl.BlockSpec((5, 32), const2),
            pl.BlockSpec((5, 32), const2),
            pl.BlockSpec((1, 32), const2),
            pl.BlockSpec((234, 128), const2),
            pl.BlockSpec((1, 128), const2),
            pl.BlockSpec((128, 32), const2),
            pl.BlockSpec((1, 32), const2),
            pl.BlockSpec((32, 1), const2),
            pl.BlockSpec((1, 1), const2),
        ],
        out_specs=pl.BlockSpec((1, 1), const2),
        out_shape=jax.ShapeDtypeStruct((1, 1), jnp.float32),
        scratch_shapes=[
            pltpu.VMEM((N, 5), jnp.float32),
            pltpu.VMEM((CHUNK, 5), jnp.float32),
        ],
    )(src3, dst3, ea3, h, h2, mol2, Ms, BeTs, cs,
      Rw1T, Rw2T, Rb2, f1w, f1b, f2w, f2b, f3w, f3b)
    return out.reshape(1)
